# baseline (speedup 1.0000x reference)
import functools

import jax
import jax.numpy as jnp
from jax import lax
from jax.experimental import pallas as pl
from jax.experimental.pallas import tpu as pltpu
from jax.experimental.pallas import tpu_sc as plsc

EMBEDDING_DIM = 64
CHUNK = 128


@functools.lru_cache(maxsize=None)
def _make_gather(num_chunks: int, dim: int):
    info = plsc.get_sparse_core_info()
    nc, ns = info.num_cores, info.num_subcores
    nw = nc * ns
    mesh = plsc.VectorSubcoreMesh(core_axis_name="c", subcore_axis_name="s")

    @functools.partial(
        pl.kernel,
        out_type=jax.ShapeDtypeStruct((nw, num_chunks, CHUNK, dim), jnp.float32),
        mesh=mesh,
        scratch_types=[
            pltpu.VMEM((num_chunks, CHUNK), jnp.int32),
            pltpu.VMEM((CHUNK, dim), jnp.float32),
            pltpu.SemaphoreType.DMA,
        ],
        compiler_params=pltpu.CompilerParams(use_tc_tiling_on_sc=False),
    )
    def gather_kernel(idx_hbm, table_hbm, out_hbm, idx_v, rows_v, sem):
        wid = lax.axis_index("c") * ns + lax.axis_index("s")
        pltpu.sync_copy(idx_hbm.at[wid], idx_v)

        def body(j, carry):
            pltpu.async_copy(table_hbm.at[idx_v.at[j]], rows_v, sem).wait()
            pltpu.sync_copy(rows_v, out_hbm.at[wid, j])
            return carry

        lax.fori_loop(0, num_chunks, body, 0)

    return gather_kernel


def kernel(token_ids, weight):
    orig_shape = token_ids.shape
    dim = weight.shape[1]
    idx = token_ids.reshape(-1).astype(jnp.int32)
    b = idx.shape[0]
    info = plsc.get_sparse_core_info()
    nw = info.num_cores * info.num_subcores
    b_per_w = b // nw
    num_chunks = b_per_w // CHUNK
    assert b == nw * num_chunks * CHUNK, (b, nw, CHUNK)
    idx3 = idx.reshape(nw, num_chunks, CHUNK)
    out = _make_gather(num_chunks, dim)(idx3, weight)
    return out.reshape(*orig_shape, dim)


# speedup vs baseline: 1.0612x; 1.0612x over previous
import functools

import jax
import jax.numpy as jnp
from jax import lax
from jax.experimental import pallas as pl
from jax.experimental.pallas import tpu as pltpu
from jax.experimental.pallas import tpu_sc as plsc

CHUNK = 128
NBUF = 8


@functools.lru_cache(maxsize=None)
def _make_gather(num_chunks: int, dim: int):
    info = plsc.get_sparse_core_info()
    nc, ns = info.num_cores, info.num_subcores
    nw = nc * ns
    num_groups = num_chunks // NBUF
    assert num_chunks == num_groups * NBUF
    mesh = plsc.VectorSubcoreMesh(core_axis_name="c", subcore_axis_name="s")

    @functools.partial(
        pl.kernel,
        out_type=jax.ShapeDtypeStruct((nw, num_chunks, CHUNK, dim), jnp.float32),
        mesh=mesh,
        scratch_types=[
            pltpu.VMEM((num_chunks, CHUNK), jnp.int32),
            pltpu.VMEM((NBUF, CHUNK, dim), jnp.float32),
            pltpu.SemaphoreType.DMA((NBUF,)),
            pltpu.SemaphoreType.DMA((NBUF,)),
        ],
        compiler_params=pltpu.CompilerParams(use_tc_tiling_on_sc=False),
    )
    def gather_kernel(idx_hbm, table_hbm, out_hbm, idx_v, bufs, gsem, ssem):
        wid = lax.axis_index("c") * ns + lax.axis_index("s")
        pltpu.sync_copy(idx_hbm.at[wid], idx_v)

        def gather(j, b):
            return pltpu.make_async_copy(
                table_hbm.at[idx_v.at[j]], bufs.at[b], gsem.at[b])

        def scatter(j, b):
            return pltpu.make_async_copy(
                bufs.at[b], out_hbm.at[wid, j], ssem.at[b])

        for b in range(NBUF):
            gather(b, b).start()

        def body(g, carry):
            base = g * NBUF
            for b in range(NBUF):
                gather(base + b, b).wait()
                scatter(base + b, b).start()
            for b in range(NBUF):
                scatter(base + b, b).wait()
                gather(base + NBUF + b, b).start()
            return carry

        lax.fori_loop(0, num_groups - 1, body, 0)

        base = (num_groups - 1) * NBUF
        for b in range(NBUF):
            gather(base + b, b).wait()
            scatter(base + b, b).start()
        for b in range(NBUF):
            scatter(base + b, b).wait()

    return gather_kernel


def kernel(token_ids, weight):
    orig_shape = token_ids.shape
    dim = weight.shape[1]
    idx = token_ids.reshape(-1).astype(jnp.int32)
    b = idx.shape[0]
    info = plsc.get_sparse_core_info()
    nw = info.num_cores * info.num_subcores
    num_chunks = b // (nw * CHUNK)
    assert b == nw * num_chunks * CHUNK, (b, nw, CHUNK)
    idx3 = idx.reshape(nw, num_chunks, CHUNK)
    out = _make_gather(num_chunks, dim)(idx3, weight)
    return out.reshape(*orig_shape, dim)
